# baseline (device time: 86204 ns/iter reference)
import jax
import jax.numpy as jnp
from jax import lax
from jax.experimental import pallas as pl
from jax.experimental.pallas import tpu as pltpu

N_DEV = 4
M_GLOBAL = 4096
CHUNK = M_GLOBAL // N_DEV
N_COLS = 2048
HALF = N_COLS // 2
S = 4
SUB = CHUNK // S


def kernel(x, w_mat):
    def body(x_hbm, w_hbm, out_ref,
             x_vmem, w_vmem, w_bf,
             send_r, recv_r, send_l, recv_l, out_stage,
             x_sems, w_sem,
             ss_r, rs_r, ss_l, rs_l, copy_sems):
        my = lax.axis_index("i")
        left = (my - 1) % N_DEV
        right = (my + 1) % N_DEV

        def mm_sub(c, s, col0):
            return jnp.dot(
                x_vmem[pl.ds(c * CHUNK + s * SUB, SUB), :].astype(jnp.bfloat16),
                w_bf[:, pl.ds(col0, HALF)],
                preferred_element_type=jnp.float32,
            )

        def make(h, s, go_right):
            if go_right:
                return pltpu.make_async_remote_copy(
                    src_ref=send_r.at[h, pl.ds(s * SUB, SUB), :],
                    dst_ref=recv_r.at[h, pl.ds(s * SUB, SUB), :],
                    send_sem=ss_r.at[h, s], recv_sem=rs_r.at[h, s],
                    device_id=(right,), device_id_type=pl.DeviceIdType.MESH,
                )
            return pltpu.make_async_remote_copy(
                src_ref=send_l.at[h, pl.ds(s * SUB, SUB), :],
                dst_ref=recv_l.at[h, pl.ds(s * SUB, SUB), :],
                send_sem=ss_l.at[h, s], recv_sem=rs_l.at[h, s],
                device_id=(left,), device_id_type=pl.DeviceIdType.MESH,
            )

        rd_r = [[make(h, s, True) for s in range(S)] for h in range(N_DEV - 1)]
        rd_l = [[make(h, s, False) for s in range(S)] for h in range(N_DEV - 1)]

        w_cp_a = pltpu.make_async_copy(
            w_hbm.at[:, pl.ds(0, HALF)], w_vmem.at[:, pl.ds(0, HALF)],
            w_sem.at[0])
        w_cp_b = pltpu.make_async_copy(
            w_hbm.at[:, pl.ds(HALF, HALF)], w_vmem.at[:, pl.ds(HALF, HALF)],
            w_sem.at[1])
        w_cp_a.start()
        w_cp_b.start()
        x_cps = []
        for k, c in enumerate(((my - 1) % N_DEV, (my + 1) % N_DEV,
                               (my + 2) % N_DEV, my)):
            cp = pltpu.make_async_copy(
                x_hbm.at[pl.ds(c * CHUNK, CHUNK), :],
                x_vmem.at[pl.ds(c * CHUNK, CHUNK), :],
                x_sems.at[k],
            )
            cp.start()
            x_cps.append(cp)

        barrier_sem = pltpu.get_barrier_semaphore()
        for nbr in (left, right):
            pl.semaphore_signal(
                barrier_sem, inc=1,
                device_id=(nbr,), device_id_type=pl.DeviceIdType.MESH,
            )

        w_cp_a.wait()
        w_bf[:, pl.ds(0, HALF)] = w_vmem[:, pl.ds(0, HALF)].astype(jnp.bfloat16)
        x_cps[0].wait()

        send_r[0, pl.ds(0, SUB), :] = (
            mm_sub((my - 1) % N_DEV, 0, 0).astype(jnp.bfloat16))
        pl.semaphore_wait(barrier_sem, 2)
        rd_r[0][0].start()
        w_cp_b.wait()
        w_bf[:, pl.ds(HALF, HALF)] = (
            w_vmem[:, pl.ds(HALF, HALF)].astype(jnp.bfloat16))
        x_cps[1].wait()
        send_l[0, pl.ds(0, SUB), :] = (
            mm_sub((my + 1) % N_DEV, 0, HALF).astype(jnp.bfloat16))
        rd_l[0][0].start()
        for s in range(1, S):
            send_r[0, pl.ds(s * SUB, SUB), :] = (
                mm_sub((my - 1) % N_DEV, s, 0).astype(jnp.bfloat16))
            rd_r[0][s].start()
            send_l[0, pl.ds(s * SUB, SUB), :] = (
                mm_sub((my + 1) % N_DEV, s, HALF).astype(jnp.bfloat16))
            rd_l[0][s].start()

        x_cps[2].wait()

        out_copies = [None, None]

        def emit_out(slot, s, col0, val):
            if out_copies[slot] is not None:
                out_copies[slot].wait()
            out_stage[slot, :, :] = val
            cp = pltpu.make_async_copy(
                out_stage.at[slot],
                out_ref.at[pl.ds(s * SUB, SUB), pl.ds(col0, HALF)],
                copy_sems.at[slot],
            )
            cp.start()
            out_copies[slot] = cp

        for h in range(N_DEV - 1):
            last = h == N_DEV - 2
            if last:
                x_cps[3].wait()
            for s in range(S):
                pa = mm_sub((my - 2 - h) % N_DEV, s, 0)
                rd_r[h][s].wait()
                acc_a = recv_r[h, pl.ds(s * SUB, SUB), :].astype(jnp.float32) + pa
                if not last:
                    send_r[h + 1, pl.ds(s * SUB, SUB), :] = acc_a.astype(jnp.bfloat16)
                    rd_r[h + 1][s].start()
                else:
                    emit_out(0, s, 0, jnp.maximum(acc_a, 0.0))

                pb = mm_sub((my + 2 + h) % N_DEV, s, HALF)
                rd_l[h][s].wait()
                acc_b = recv_l[h, pl.ds(s * SUB, SUB), :].astype(jnp.float32) + pb
                if not last:
                    send_l[h + 1, pl.ds(s * SUB, SUB), :] = acc_b.astype(jnp.bfloat16)
                    rd_l[h + 1][s].start()
                else:
                    emit_out(1, s, HALF, jnp.maximum(acc_b, 0.0))

        for cp in out_copies:
            cp.wait()

    return pl.pallas_call(
        body,
        out_shape=jax.ShapeDtypeStruct((CHUNK, N_COLS), jnp.float32),
        in_specs=[
            pl.BlockSpec(memory_space=pl.ANY),
            pl.BlockSpec(memory_space=pl.ANY),
        ],
        out_specs=pl.BlockSpec(memory_space=pl.ANY),
        scratch_shapes=[
            pltpu.VMEM((M_GLOBAL, 1024), jnp.float32),
            pltpu.VMEM((1024, N_COLS), jnp.float32),
            pltpu.VMEM((1024, N_COLS), jnp.bfloat16),
            pltpu.VMEM((N_DEV - 1, CHUNK, HALF), jnp.bfloat16),
            pltpu.VMEM((N_DEV - 1, CHUNK, HALF), jnp.bfloat16),
            pltpu.VMEM((N_DEV - 1, CHUNK, HALF), jnp.bfloat16),
            pltpu.VMEM((N_DEV - 1, CHUNK, HALF), jnp.bfloat16),
            pltpu.VMEM((2, SUB, HALF), jnp.float32),
            pltpu.SemaphoreType.DMA((4,)),
            pltpu.SemaphoreType.DMA((2,)),
            pltpu.SemaphoreType.DMA((N_DEV - 1, S)),
            pltpu.SemaphoreType.DMA((N_DEV - 1, S)),
            pltpu.SemaphoreType.DMA((N_DEV - 1, S)),
            pltpu.SemaphoreType.DMA((N_DEV - 1, S)),
            pltpu.SemaphoreType.DMA((2,)),
        ],
        compiler_params=pltpu.CompilerParams(
            collective_id=0,
            vmem_limit_bytes=100 * 1024 * 1024,
        ),
    )(x, w_mat)


# device time: 86020 ns/iter; 1.0021x vs baseline; 1.0021x over previous
import jax
import jax.numpy as jnp
from jax import lax
from jax.experimental import pallas as pl
from jax.experimental.pallas import tpu as pltpu

N_DEV = 4
M_GLOBAL = 4096
CHUNK = M_GLOBAL // N_DEV
N_COLS = 2048
HALF = N_COLS // 2
S = 8
SUB = CHUNK // S


def kernel(x, w_mat):
    def body(x_hbm, w_hbm, out_ref,
             x_vmem, w_vmem, w_bf,
             send_r, recv_r, send_l, recv_l, out_stage,
             x_sems, w_sem,
             ss_r, rs_r, ss_l, rs_l, copy_sems):
        my = lax.axis_index("i")
        left = (my - 1) % N_DEV
        right = (my + 1) % N_DEV

        def mm_sub(c, s, col0):
            return jnp.dot(
                x_vmem[pl.ds(c * CHUNK + s * SUB, SUB), :].astype(jnp.bfloat16),
                w_bf[:, pl.ds(col0, HALF)],
                preferred_element_type=jnp.float32,
            )

        def make(h, s, go_right):
            if go_right:
                return pltpu.make_async_remote_copy(
                    src_ref=send_r.at[h, pl.ds(s * SUB, SUB), :],
                    dst_ref=recv_r.at[h, pl.ds(s * SUB, SUB), :],
                    send_sem=ss_r.at[h, s], recv_sem=rs_r.at[h, s],
                    device_id=(right,), device_id_type=pl.DeviceIdType.MESH,
                )
            return pltpu.make_async_remote_copy(
                src_ref=send_l.at[h, pl.ds(s * SUB, SUB), :],
                dst_ref=recv_l.at[h, pl.ds(s * SUB, SUB), :],
                send_sem=ss_l.at[h, s], recv_sem=rs_l.at[h, s],
                device_id=(left,), device_id_type=pl.DeviceIdType.MESH,
            )

        rd_r = [[make(h, s, True) for s in range(S)] for h in range(N_DEV - 1)]
        rd_l = [[make(h, s, False) for s in range(S)] for h in range(N_DEV - 1)]

        w_cp_a = pltpu.make_async_copy(
            w_hbm.at[:, pl.ds(0, HALF)], w_vmem.at[:, pl.ds(0, HALF)],
            w_sem.at[0])
        w_cp_b = pltpu.make_async_copy(
            w_hbm.at[:, pl.ds(HALF, HALF)], w_vmem.at[:, pl.ds(HALF, HALF)],
            w_sem.at[1])
        w_cp_a.start()
        w_cp_b.start()
        x_cps = []
        for k, c in enumerate(((my - 1) % N_DEV, (my + 1) % N_DEV,
                               (my + 2) % N_DEV, my)):
            cp = pltpu.make_async_copy(
                x_hbm.at[pl.ds(c * CHUNK, CHUNK), :],
                x_vmem.at[pl.ds(c * CHUNK, CHUNK), :],
                x_sems.at[k],
            )
            cp.start()
            x_cps.append(cp)

        barrier_sem = pltpu.get_barrier_semaphore()
        for nbr in (left, right):
            pl.semaphore_signal(
                barrier_sem, inc=1,
                device_id=(nbr,), device_id_type=pl.DeviceIdType.MESH,
            )

        w_cp_a.wait()
        w_bf[:, pl.ds(0, HALF)] = w_vmem[:, pl.ds(0, HALF)].astype(jnp.bfloat16)
        x_cps[0].wait()

        send_r[0, pl.ds(0, SUB), :] = (
            mm_sub((my - 1) % N_DEV, 0, 0).astype(jnp.bfloat16))
        pl.semaphore_wait(barrier_sem, 2)
        rd_r[0][0].start()
        w_cp_b.wait()
        w_bf[:, pl.ds(HALF, HALF)] = (
            w_vmem[:, pl.ds(HALF, HALF)].astype(jnp.bfloat16))
        x_cps[1].wait()
        send_l[0, pl.ds(0, SUB), :] = (
            mm_sub((my + 1) % N_DEV, 0, HALF).astype(jnp.bfloat16))
        rd_l[0][0].start()
        for s in range(1, S):
            send_r[0, pl.ds(s * SUB, SUB), :] = (
                mm_sub((my - 1) % N_DEV, s, 0).astype(jnp.bfloat16))
            rd_r[0][s].start()
            send_l[0, pl.ds(s * SUB, SUB), :] = (
                mm_sub((my + 1) % N_DEV, s, HALF).astype(jnp.bfloat16))
            rd_l[0][s].start()

        x_cps[2].wait()

        out_copies = [None, None]

        def emit_out(slot, s, col0, val):
            if out_copies[slot] is not None:
                out_copies[slot].wait()
            out_stage[slot, :, :] = val
            cp = pltpu.make_async_copy(
                out_stage.at[slot],
                out_ref.at[pl.ds(s * SUB, SUB), pl.ds(col0, HALF)],
                copy_sems.at[slot],
            )
            cp.start()
            out_copies[slot] = cp

        for h in range(N_DEV - 1):
            last = h == N_DEV - 2
            if last:
                x_cps[3].wait()
            for s in range(S):
                pa = mm_sub((my - 2 - h) % N_DEV, s, 0)
                rd_r[h][s].wait()
                acc_a = recv_r[h, pl.ds(s * SUB, SUB), :].astype(jnp.float32) + pa
                if not last:
                    send_r[h + 1, pl.ds(s * SUB, SUB), :] = acc_a.astype(jnp.bfloat16)
                    rd_r[h + 1][s].start()
                else:
                    emit_out(0, s, 0, jnp.maximum(acc_a, 0.0))

                pb = mm_sub((my + 2 + h) % N_DEV, s, HALF)
                rd_l[h][s].wait()
                acc_b = recv_l[h, pl.ds(s * SUB, SUB), :].astype(jnp.float32) + pb
                if not last:
                    send_l[h + 1, pl.ds(s * SUB, SUB), :] = acc_b.astype(jnp.bfloat16)
                    rd_l[h + 1][s].start()
                else:
                    emit_out(1, s, HALF, jnp.maximum(acc_b, 0.0))

        for cp in out_copies:
            cp.wait()

    return pl.pallas_call(
        body,
        out_shape=jax.ShapeDtypeStruct((CHUNK, N_COLS), jnp.float32),
        in_specs=[
            pl.BlockSpec(memory_space=pl.ANY),
            pl.BlockSpec(memory_space=pl.ANY),
        ],
        out_specs=pl.BlockSpec(memory_space=pl.ANY),
        scratch_shapes=[
            pltpu.VMEM((M_GLOBAL, 1024), jnp.float32),
            pltpu.VMEM((1024, N_COLS), jnp.float32),
            pltpu.VMEM((1024, N_COLS), jnp.bfloat16),
            pltpu.VMEM((N_DEV - 1, CHUNK, HALF), jnp.bfloat16),
            pltpu.VMEM((N_DEV - 1, CHUNK, HALF), jnp.bfloat16),
            pltpu.VMEM((N_DEV - 1, CHUNK, HALF), jnp.bfloat16),
            pltpu.VMEM((N_DEV - 1, CHUNK, HALF), jnp.bfloat16),
            pltpu.VMEM((2, SUB, HALF), jnp.float32),
            pltpu.SemaphoreType.DMA((4,)),
            pltpu.SemaphoreType.DMA((2,)),
            pltpu.SemaphoreType.DMA((N_DEV - 1, S)),
            pltpu.SemaphoreType.DMA((N_DEV - 1, S)),
            pltpu.SemaphoreType.DMA((N_DEV - 1, S)),
            pltpu.SemaphoreType.DMA((N_DEV - 1, S)),
            pltpu.SemaphoreType.DMA((2,)),
        ],
        compiler_params=pltpu.CompilerParams(
            collective_id=0,
            vmem_limit_bytes=100 * 1024 * 1024,
        ),
    )(x, w_mat)
